# baseline (speedup 1.0000x reference)
import functools

import jax
import jax.numpy as jnp
from jax import lax
from jax.experimental import pallas as pl
from jax.experimental.pallas import tpu as pltpu
from jax.experimental.pallas import tpu_sc as plsc

_VOCAB = 2048
_HIDDEN = 32
_BATCH = 4
_SEQ = 8192

_NUM_CORES = 2
_NUM_SUBCORES = 16
_NW = _NUM_CORES * _NUM_SUBCORES
_TOTAL = _BATCH * _SEQ
_PER_W = _TOTAL // _NW
_CHUNK = 128
_NCH = _PER_W // _CHUNK
_W_PER_B = _NW // _BATCH
_S_PER_W = _SEQ // _W_PER_B
_LANES = 16


def _proj_body(table_ref, w_ref, b_ref, out_ref):
    out_ref[...] = lax.dot_general(
        table_ref[...], w_ref[...],
        dimension_numbers=(((1,), (1,)), ((), ())),
        preferred_element_type=jnp.float32,
    ) + b_ref[...]


def _project_table(embed_table, proj_W, proj_b):
    return pl.pallas_call(
        _proj_body,
        out_shape=jax.ShapeDtypeStruct((_VOCAB, _HIDDEN), jnp.float32),
    )(embed_table, proj_W, proj_b.reshape(1, _HIDDEN))


_sc_mesh = plsc.VectorSubcoreMesh(
    core_axis_name="c", subcore_axis_name="s",
    num_cores=_NUM_CORES, num_subcores=_NUM_SUBCORES,
)


@functools.partial(
    pl.kernel,
    out_type=jax.ShapeDtypeStruct(
        (_BATCH, _HIDDEN // 8, _SEQ // 128, 8, 128), jnp.float32),
    mesh=_sc_mesh,
    scratch_types=[
        pltpu.VMEM((_PER_W,), jnp.int32),
        pltpu.VMEM((_NCH, _CHUNK, _HIDDEN), jnp.float32),
        pltpu.VMEM((_NCH, _HIDDEN, _CHUNK), jnp.float32),
        pltpu.SemaphoreType.DMA,
        pltpu.SemaphoreType.DMA,
    ],
    compiler_params=pltpu.CompilerParams(
        use_tc_tiling_on_sc=False, needs_layout_passes=False),
)
def _sc_gather(ids_hbm, table_hbm, out_hbm, idx_v, rows_v, t_v, sem_g, sem_w):
    wid = lax.axis_index("s") * _NUM_CORES + lax.axis_index("c")
    b = wid // _W_PER_B
    j_base = (wid % _W_PER_B) * _NCH
    s0 = (wid % _W_PER_B) * _S_PER_W
    pltpu.sync_copy(ids_hbm.at[b, pl.ds(s0, _S_PER_W)], idx_v)
    gathers = [
        pltpu.async_copy(
            table_hbm.at[idx_v.at[pl.ds(j * _CHUNK, _CHUNK)]],
            rows_v.at[j], sem_g)
        for j in range(_NCH)
    ]
    lane_iota = lax.iota(jnp.int32, _LANES)
    writes = []
    for j in range(_NCH):
        gathers[j].wait()

        for h in range(_HIDDEN):
            col = jnp.full((_LANES,), h, jnp.int32)
            for g in range(_CHUNK // _LANES):
                vec = plsc.load_gather(
                    rows_v.at[j], [lane_iota + (g * _LANES), col])
                t_v[j, h, pl.ds(g * _LANES, _LANES)] = vec

        for i in range(_HIDDEN // 8):
            writes.append(pltpu.async_copy(
                t_v.at[j, pl.ds(8 * i, 8)],
                out_hbm.at[b, i, j_base + j],
                sem_w))
    for w in writes:
        w.wait()


def kernel(input_ids, attention_mask, return_dict, embed_table, proj_W, proj_b):
    del attention_mask, return_dict
    projected = _project_table(embed_table, proj_W, proj_b)
    out5 = _sc_gather(input_ids.astype(jnp.int32), projected)
    return out5.transpose(0, 2, 4, 1, 3).reshape(_BATCH, _SEQ, _HIDDEN)


# speedup vs baseline: 1.3510x; 1.3510x over previous
import functools

import jax
import jax.numpy as jnp
from jax import lax
from jax.experimental import pallas as pl
from jax.experimental.pallas import tpu as pltpu
from jax.experimental.pallas import tpu_sc as plsc

_VOCAB = 2048
_HIDDEN = 32
_BATCH = 4
_SEQ = 8192

_NUM_CORES = 2
_NUM_SUBCORES = 16
_NW = _NUM_CORES * _NUM_SUBCORES
_TOTAL = _BATCH * _SEQ
_PER_W = _TOTAL // _NW
_CHUNK = 128
_NCH = _PER_W // _CHUNK

_SBLK = 2048
_NSB = _SEQ // _SBLK
_ROWS = _TOTAL * _HIDDEN // 128
_RBLK = _SBLK * _HIDDEN // 128

_sc_mesh = plsc.VectorSubcoreMesh(
    core_axis_name="c", subcore_axis_name="s",
    num_cores=_NUM_CORES, num_subcores=_NUM_SUBCORES,
)


@functools.partial(
    pl.kernel,
    out_type=jax.ShapeDtypeStruct((_TOTAL, _HIDDEN), jnp.float32),
    mesh=_sc_mesh,
    scratch_types=[
        pltpu.VMEM((_PER_W,), jnp.int32),
        pltpu.VMEM((_NCH, _CHUNK, _HIDDEN), jnp.float32),
        pltpu.SemaphoreType.DMA,
        pltpu.SemaphoreType.DMA,
    ],
    compiler_params=pltpu.CompilerParams(use_tc_tiling_on_sc=False),
)
def _sc_gather(ids_hbm, table_hbm, out_hbm, idx_v, rows_v, sem_g, sem_w):
    wid = lax.axis_index("s") * _NUM_CORES + lax.axis_index("c")
    base = wid * _PER_W
    pltpu.sync_copy(ids_hbm.at[pl.ds(base, _PER_W)], idx_v)
    gathers = [
        pltpu.async_copy(
            table_hbm.at[idx_v.at[pl.ds(j * _CHUNK, _CHUNK)]],
            rows_v.at[j], sem_g)
        for j in range(_NCH)
    ]
    writes = []
    for j in range(_NCH):
        gathers[j].wait()
        writes.append(pltpu.async_copy(
            rows_v.at[j],
            out_hbm.at[pl.ds(base + j * _CHUNK, _CHUNK)],
            sem_w))
    for w in writes:
        w.wait()


def _proj_body(g_ref, w4_ref, b4_ref, out_ref):
    y = jnp.dot(
        g_ref[...], w4_ref[...], preferred_element_type=jnp.float32,
    ) + b4_ref[...]
    yt = y.T
    out_ref[...] = jnp.concatenate(
        [yt[32 * q:32 * (q + 1)] for q in range(4)], axis=1,
    )[None]


def _project_packed(g, proj_W, proj_b):
    w4 = jnp.kron(jnp.eye(4, dtype=jnp.float32), proj_W.T)
    b4 = jnp.tile(proj_b, 4)[None]
    return pl.pallas_call(
        _proj_body,
        grid=(_BATCH, _NSB),
        in_specs=[
            pl.BlockSpec((_RBLK, 128), lambda b, sb: (b * _NSB + sb, 0)),
            pl.BlockSpec((128, 128), lambda b, sb: (0, 0)),
            pl.BlockSpec((1, 128), lambda b, sb: (0, 0)),
        ],
        out_specs=pl.BlockSpec((1, _HIDDEN, _SBLK), lambda b, sb: (b, 0, sb)),
        out_shape=jax.ShapeDtypeStruct((_BATCH, _HIDDEN, _SEQ), jnp.float32),
    )(g, w4, b4)


def kernel(input_ids, attention_mask, return_dict, embed_table, proj_W, proj_b):
    del attention_mask, return_dict
    ids_p = (input_ids.astype(jnp.int32)
             .reshape(_BATCH, _NSB, 4, _SBLK // 4)
             .transpose(0, 1, 3, 2)
             .reshape(_TOTAL))
    g = _sc_gather(ids_p, embed_table)
    yt = _project_packed(g.reshape(_ROWS, 128), proj_W, proj_b)
    return yt.transpose(0, 2, 1)


# speedup vs baseline: 1.5947x; 1.1804x over previous
import functools

import jax
import jax.numpy as jnp
from jax import lax
from jax.experimental import pallas as pl
from jax.experimental.pallas import tpu as pltpu
from jax.experimental.pallas import tpu_sc as plsc

_VOCAB = 2048
_HIDDEN = 32
_BATCH = 4
_SEQ = 8192

_NUM_CORES = 2
_NUM_SUBCORES = 16
_NW = _NUM_CORES * _NUM_SUBCORES
_TOTAL = _BATCH * _SEQ
_PER_W = _TOTAL // _NW
_CHUNK = 128
_NCH = _PER_W // _CHUNK

_SBLK = 8192
_NSB = _SEQ // _SBLK
_ROWS = _TOTAL * _HIDDEN // 128
_RBLK = _SBLK * _HIDDEN // 128

_sc_mesh = plsc.VectorSubcoreMesh(
    core_axis_name="c", subcore_axis_name="s",
    num_cores=_NUM_CORES, num_subcores=_NUM_SUBCORES,
)


@functools.partial(
    pl.kernel,
    out_type=jax.ShapeDtypeStruct((_TOTAL, _HIDDEN), jnp.float32),
    mesh=_sc_mesh,
    scratch_types=[
        pltpu.VMEM((_PER_W,), jnp.int32),
        pltpu.VMEM((_NCH, _CHUNK, _HIDDEN), jnp.float32),
        pltpu.SemaphoreType.DMA,
        pltpu.SemaphoreType.DMA,
    ],
    compiler_params=pltpu.CompilerParams(use_tc_tiling_on_sc=False),
)
def _sc_gather(ids_hbm, table_hbm, out_hbm, idx_v, rows_v, sem_g, sem_w):
    wid = lax.axis_index("s") * _NUM_CORES + lax.axis_index("c")
    base = wid * _PER_W
    pltpu.sync_copy(ids_hbm.at[wid], idx_v)
    gathers = [
        pltpu.async_copy(
            table_hbm.at[idx_v.at[pl.ds(j * _CHUNK, _CHUNK)]],
            rows_v.at[j], sem_g)
        for j in range(_NCH)
    ]
    writes = []
    for j in range(_NCH):
        gathers[j].wait()
        writes.append(pltpu.async_copy(
            rows_v.at[j],
            out_hbm.at[pl.ds(base + j * _CHUNK, _CHUNK)],
            sem_w))
    for w in writes:
        w.wait()


def _proj_body(g_ref, w4_ref, b4_ref, out_ref):
    y = jnp.dot(
        g_ref[...], w4_ref[...], preferred_element_type=jnp.float32,
    ) + b4_ref[...]
    yt = y.T
    out_ref[...] = jnp.concatenate(
        [yt[32 * q:32 * (q + 1)] for q in range(4)], axis=1,
    )[None]


def _project_packed(g, proj_W, proj_b):
    w4 = jnp.kron(jnp.eye(4, dtype=jnp.float32), proj_W.T)
    b4 = jnp.tile(proj_b, 4)[None]
    return pl.pallas_call(
        _proj_body,
        grid=(_BATCH, _NSB),
        in_specs=[
            pl.BlockSpec((_RBLK, 128), lambda b, sb: (b * _NSB + sb, 0)),
            pl.BlockSpec((128, 128), lambda b, sb: (0, 0)),
            pl.BlockSpec((1, 128), lambda b, sb: (0, 0)),
        ],
        out_specs=pl.BlockSpec((1, _HIDDEN, _SBLK), lambda b, sb: (b, 0, sb)),
        out_shape=jax.ShapeDtypeStruct((_BATCH, _HIDDEN, _SEQ), jnp.float32),
    )(g, w4, b4)


def kernel(input_ids, attention_mask, return_dict, embed_table, proj_W, proj_b):
    del attention_mask, return_dict
    ids_p = (input_ids.astype(jnp.int32)
             .reshape(_BATCH, _NSB, 4, _SBLK // 4)
             .transpose(0, 1, 3, 2)
             .reshape(_NW, _PER_W))
    g = _sc_gather(ids_p, embed_table)
    yt = _project_packed(g.reshape(_ROWS, 128), proj_W, proj_b)
    return yt.transpose(0, 2, 1)


# speedup vs baseline: 1.7139x; 1.0747x over previous
import functools

import jax
import jax.numpy as jnp
from jax import lax
from jax.experimental import pallas as pl
from jax.experimental.pallas import tpu as pltpu
from jax.experimental.pallas import tpu_sc as plsc

_VOCAB = 2048
_HIDDEN = 32
_BATCH = 4
_SEQ = 8192

_NUM_CORES = 2
_NUM_SUBCORES = 16
_NW = _NUM_CORES * _NUM_SUBCORES
_TOTAL = _BATCH * _SEQ
_PER_W = _TOTAL // _NW
_CHUNK = 128
_NCH = _PER_W // _CHUNK

_W_PER_B = _NW // _BATCH
_SBLK = 8192
_NSB = _SEQ // _SBLK
_ROWS = _TOTAL * _HIDDEN // 128
_RBLK = _SBLK * _HIDDEN // 128

_sc_mesh = plsc.VectorSubcoreMesh(
    core_axis_name="c", subcore_axis_name="s",
    num_cores=_NUM_CORES, num_subcores=_NUM_SUBCORES,
)


@functools.partial(
    pl.kernel,
    out_type=jax.ShapeDtypeStruct((_TOTAL, _HIDDEN), jnp.float32),
    mesh=_sc_mesh,
    scratch_types=[
        pltpu.VMEM((_PER_W,), jnp.int32),
        pltpu.VMEM((_NCH, _CHUNK), jnp.int32),
        pltpu.VMEM((_NCH, _CHUNK, _HIDDEN), jnp.float32),
        pltpu.SemaphoreType.DMA,
        pltpu.SemaphoreType.DMA,
    ],
    compiler_params=pltpu.CompilerParams(
        use_tc_tiling_on_sc=False, needs_layout_passes=False),
)
def _sc_gather(ids_hbm, table_hbm, out_hbm, ids4_v, idx_v, rows_v, sem_g, sem_w):
    wid = lax.axis_index("s") * _NUM_CORES + lax.axis_index("c")
    base = wid * _PER_W
    b = wid // _W_PER_B
    k0 = (wid % _W_PER_B) * (_PER_W // 4)
    for q in range(4):
        pltpu.sync_copy(
            ids_hbm.at[b, pl.ds(q * (_SEQ // 4) + k0, _PER_W // 4)],
            ids4_v.at[pl.ds(q * (_PER_W // 4), _PER_W // 4)])
    lane = lax.iota(jnp.int32, 16)
    base_pat = (lane & 3) * (_PER_W // 4) + (lane >> 2)
    for u in range(_PER_W // 16):
        vec = plsc.load_gather(ids4_v, [base_pat + (4 * u)])
        idx_v[u // 8, pl.ds((u % 8) * 16, 16)] = vec
    gathers = [
        pltpu.async_copy(
            table_hbm.at[idx_v.at[j]],
            rows_v.at[j], sem_g)
        for j in range(_NCH)
    ]
    writes = []
    for j in range(_NCH):
        gathers[j].wait()
        writes.append(pltpu.async_copy(
            rows_v.at[j],
            out_hbm.at[pl.ds(base + j * _CHUNK, _CHUNK)],
            sem_w))
    for w in writes:
        w.wait()


def _proj_body(g_ref, w4_ref, b4_ref, out_ref):
    y = jnp.dot(
        g_ref[...], w4_ref[...], preferred_element_type=jnp.float32,
    ) + b4_ref[...]
    yt = y.T
    out_ref[...] = jnp.concatenate(
        [yt[32 * q:32 * (q + 1)] for q in range(4)], axis=1,
    )[None]


def _project_packed(g, proj_W, proj_b):
    w4 = jnp.kron(jnp.eye(4, dtype=jnp.float32), proj_W.T)
    b4 = jnp.tile(proj_b, 4)[None]
    return pl.pallas_call(
        _proj_body,
        grid=(_BATCH, _NSB),
        in_specs=[
            pl.BlockSpec((_RBLK, 128), lambda b, sb: (b * _NSB + sb, 0)),
            pl.BlockSpec((128, 128), lambda b, sb: (0, 0)),
            pl.BlockSpec((1, 128), lambda b, sb: (0, 0)),
        ],
        out_specs=pl.BlockSpec((1, _HIDDEN, _SBLK), lambda b, sb: (b, 0, sb)),
        out_shape=jax.ShapeDtypeStruct((_BATCH, _HIDDEN, _SEQ), jnp.float32),
    )(g, w4, b4)


def kernel(input_ids, attention_mask, return_dict, embed_table, proj_W, proj_b):
    del attention_mask, return_dict
    g = _sc_gather(input_ids.astype(jnp.int32), embed_table)
    yt = _project_packed(g.reshape(_ROWS, 128), proj_W, proj_b)
    return yt.transpose(0, 2, 1)


# speedup vs baseline: 1.7155x; 1.0009x over previous
import functools

import jax
import jax.numpy as jnp
from jax import lax
from jax.experimental import pallas as pl
from jax.experimental.pallas import tpu as pltpu
from jax.experimental.pallas import tpu_sc as plsc

_VOCAB = 2048
_HIDDEN = 32
_BATCH = 4
_SEQ = 8192

_NUM_CORES = 2
_NUM_SUBCORES = 16
_NW = _NUM_CORES * _NUM_SUBCORES
_TOTAL = _BATCH * _SEQ
_PER_W = _TOTAL // _NW
_CHUNK = 128
_NCH = _PER_W // _CHUNK

_W_PER_B = _NW // _BATCH
_SBLK = 8192
_NSB = _SEQ // _SBLK
_ROWS = _TOTAL * _HIDDEN // 128
_RBLK = _SBLK * _HIDDEN // 128

_sc_mesh = plsc.VectorSubcoreMesh(
    core_axis_name="c", subcore_axis_name="s",
    num_cores=_NUM_CORES, num_subcores=_NUM_SUBCORES,
)


@functools.partial(
    pl.kernel,
    out_type=jax.ShapeDtypeStruct((_TOTAL, _HIDDEN), jnp.float32),
    mesh=_sc_mesh,
    scratch_types=[
        pltpu.VMEM((4 * 264,), jnp.int32),
        pltpu.VMEM((_NCH, _CHUNK), jnp.int32),
        pltpu.VMEM((_NCH, _CHUNK, _HIDDEN), jnp.float32),
        pltpu.SemaphoreType.DMA,
        pltpu.SemaphoreType.DMA,
    ],
    compiler_params=pltpu.CompilerParams(
        use_tc_tiling_on_sc=False, needs_layout_passes=False),
)
def _sc_gather(ids_hbm, table_hbm, out_hbm, ids4_v, idx_v, rows_v, sem_g, sem_w):
    wid = lax.axis_index("s") * _NUM_CORES + lax.axis_index("c")
    base = wid * _PER_W
    b = wid // _W_PER_B
    k0 = (wid % _W_PER_B) * (_PER_W // 4)
    for q in range(4):
        pltpu.sync_copy(
            ids_hbm.at[b, pl.ds(q * (_SEQ // 4) + k0, _PER_W // 4)],
            ids4_v.at[pl.ds(q * 264, _PER_W // 4)])
    lane = lax.iota(jnp.int32, 16)
    base_pat = (lane & 3) * 264 + (lane >> 2)
    for u in range(_PER_W // 16):
        vec = plsc.load_gather(ids4_v, [base_pat + (4 * u)])
        idx_v[u // 8, pl.ds((u % 8) * 16, 16)] = vec
    gathers = [
        pltpu.async_copy(
            table_hbm.at[idx_v.at[j]],
            rows_v.at[j], sem_g)
        for j in range(_NCH)
    ]
    writes = []
    for j in range(_NCH):
        gathers[j].wait()
        writes.append(pltpu.async_copy(
            rows_v.at[j],
            out_hbm.at[pl.ds(base + j * _CHUNK, _CHUNK)],
            sem_w))
    for w in writes:
        w.wait()


def _proj_body(g_ref, w4_ref, b4_ref, out_ref):
    y = jnp.dot(
        g_ref[...], w4_ref[...], preferred_element_type=jnp.float32,
    ) + b4_ref[...]
    yt = y.T
    out_ref[...] = jnp.concatenate(
        [yt[32 * q:32 * (q + 1)] for q in range(4)], axis=1,
    )[None]


def _project_packed(g, proj_W, proj_b):
    w4 = jnp.kron(jnp.eye(4, dtype=jnp.float32), proj_W.T)
    b4 = jnp.tile(proj_b, 4)[None]
    return pl.pallas_call(
        _proj_body,
        grid=(_BATCH, _NSB),
        in_specs=[
            pl.BlockSpec((_RBLK, 128), lambda b, sb: (b * _NSB + sb, 0)),
            pl.BlockSpec((128, 128), lambda b, sb: (0, 0)),
            pl.BlockSpec((1, 128), lambda b, sb: (0, 0)),
        ],
        out_specs=pl.BlockSpec((1, _HIDDEN, _SBLK), lambda b, sb: (b, 0, sb)),
        out_shape=jax.ShapeDtypeStruct((_BATCH, _HIDDEN, _SEQ), jnp.float32),
    )(g, w4, b4)


def kernel(input_ids, attention_mask, return_dict, embed_table, proj_W, proj_b):
    del attention_mask, return_dict
    g = _sc_gather(input_ids.astype(jnp.int32), embed_table)
    yt = _project_packed(g.reshape(_ROWS, 128), proj_W, proj_b)
    return yt.transpose(0, 2, 1)


# speedup vs baseline: 1.7971x; 1.0476x over previous
import functools

import jax
import jax.numpy as jnp
from jax import lax
from jax.experimental import pallas as pl
from jax.experimental.pallas import tpu as pltpu
from jax.experimental.pallas import tpu_sc as plsc

_VOCAB = 2048
_HIDDEN = 32
_BATCH = 4
_SEQ = 8192

_NUM_CORES = 2
_NUM_SUBCORES = 16
_NW = _NUM_CORES * _NUM_SUBCORES
_TOTAL = _BATCH * _SEQ
_PER_W = _TOTAL // _NW
_CHUNK = 128
_NCH = _PER_W // _CHUNK

_W_PER_B = _NW // _BATCH
_SBLK = 8192
_NSB = _SEQ // _SBLK
_ROWS = _TOTAL * _HIDDEN // 128
_RBLK = _SBLK * _HIDDEN // 128

_sc_mesh = plsc.VectorSubcoreMesh(
    core_axis_name="c", subcore_axis_name="s",
    num_cores=_NUM_CORES, num_subcores=_NUM_SUBCORES,
)


@functools.partial(
    pl.kernel,
    out_type=jax.ShapeDtypeStruct((_TOTAL, _HIDDEN), jnp.float32),
    mesh=_sc_mesh,
    scratch_types=[
        pltpu.VMEM((4 * 264,), jnp.int32),
        pltpu.VMEM((_NCH, _CHUNK), jnp.int32),
        pltpu.VMEM((_NCH, _CHUNK, _HIDDEN), jnp.float32),
        pltpu.SemaphoreType.DMA,
        pltpu.SemaphoreType.DMA,
    ],
    compiler_params=pltpu.CompilerParams(
        use_tc_tiling_on_sc=False, needs_layout_passes=False),
)
def _sc_gather(ids_hbm, table_hbm, out_hbm, ids4_v, idx_v, rows_v, sem_g, sem_w):
    wid = lax.axis_index("s") * _NUM_CORES + lax.axis_index("c")
    base = wid * _PER_W
    b = wid // _W_PER_B
    k0 = (wid % _W_PER_B) * (_PER_W // 4)
    id_copies = [
        pltpu.async_copy(
            ids_hbm.at[b, pl.ds(q * (_SEQ // 4) + k0, _PER_W // 4)],
            ids4_v.at[pl.ds(q * 264, _PER_W // 4)], sem_w)
        for q in range(4)
    ]
    for c in id_copies:
        c.wait()
    lane = lax.iota(jnp.int32, 16)
    base_pat = (lane & 3) * 264 + (lane >> 2)
    gathers = []
    for j in range(_NCH):
        for v in range(8):
            u = j * 8 + v
            vec = plsc.load_gather(ids4_v, [base_pat + (4 * u)])
            idx_v[j, pl.ds(v * 16, 16)] = vec
        gathers.append(pltpu.async_copy(
            table_hbm.at[idx_v.at[j]],
            rows_v.at[j], sem_g))
    writes = []
    for j in range(_NCH):
        gathers[j].wait()
        writes.append(pltpu.async_copy(
            rows_v.at[j],
            out_hbm.at[pl.ds(base + j * _CHUNK, _CHUNK)],
            sem_w))
    for w in writes:
        w.wait()


def _proj_body(g_ref, w4_ref, b4_ref, out_ref):
    y = jnp.dot(
        g_ref[...], w4_ref[...], preferred_element_type=jnp.float32,
    ) + b4_ref[...]
    yt = y.T
    out_ref[...] = jnp.concatenate(
        [yt[32 * q:32 * (q + 1)] for q in range(4)], axis=1,
    )[None]


def _project_packed(g, proj_W, proj_b):
    w4 = jnp.kron(jnp.eye(4, dtype=jnp.float32), proj_W.T)
    b4 = jnp.tile(proj_b, 4)[None]
    return pl.pallas_call(
        _proj_body,
        grid=(_BATCH, _NSB),
        in_specs=[
            pl.BlockSpec((_RBLK, 128), lambda b, sb: (b * _NSB + sb, 0)),
            pl.BlockSpec((128, 128), lambda b, sb: (0, 0)),
            pl.BlockSpec((1, 128), lambda b, sb: (0, 0)),
        ],
        out_specs=pl.BlockSpec((1, _HIDDEN, _SBLK), lambda b, sb: (b, 0, sb)),
        out_shape=jax.ShapeDtypeStruct((_BATCH, _HIDDEN, _SEQ), jnp.float32),
    )(g, w4, b4)


def kernel(input_ids, attention_mask, return_dict, embed_table, proj_W, proj_b):
    del attention_mask, return_dict
    g = _sc_gather(input_ids.astype(jnp.int32), embed_table)
    yt = _project_packed(g.reshape(_ROWS, 128), proj_W, proj_b)
    return yt.transpose(0, 2, 1)
